# baseline (speedup 1.0000x reference)
import jax
import jax.numpy as jnp
from jax.experimental import pallas as pl
from jax.experimental.pallas import tpu as pltpu

N = 4096
D = 128
BM = 512


def _fused_kernel(adj_low_ref, adj_high_ref, x_ref, wl_ref, wh_ref, wm_ref,
                  avl_ref, avh_ref, avm_ref, att_ref, out_ref):
    i = pl.program_id(0)
    x = x_ref[...]
    ax_l = jnp.dot(adj_low_ref[...], x, preferred_element_type=jnp.float32)
    ax_h = jnp.dot(adj_high_ref[...], x, preferred_element_type=jnp.float32)
    o_l = jax.nn.relu(jnp.dot(ax_l, wl_ref[...],
                              preferred_element_type=jnp.float32))
    o_h = jax.nn.relu(jnp.dot(ax_h, wh_ref[...],
                              preferred_element_type=jnp.float32))
    x_blk = x_ref[pl.ds(i * BM, BM), :]
    o_m = jax.nn.relu(jnp.dot(x_blk, wm_ref[...],
                              preferred_element_type=jnp.float32))

    f_l = jnp.sum(o_l * avl_ref[...], axis=1, keepdims=True)
    f_h = jnp.sum(o_h * avh_ref[...], axis=1, keepdims=True)
    f_m = jnp.sum(o_m * avm_ref[...], axis=1, keepdims=True)
    s_l = jax.nn.sigmoid(f_l)
    s_h = jax.nn.sigmoid(f_h)
    s_m = jax.nn.sigmoid(f_m)
    t_inv = 1.0 / 3.0
    l0 = (s_l * att_ref[0, 0] + s_h * att_ref[1, 0] + s_m * att_ref[2, 0]) * t_inv
    l1 = (s_l * att_ref[0, 1] + s_h * att_ref[1, 1] + s_m * att_ref[2, 1]) * t_inv
    l2 = (s_l * att_ref[0, 2] + s_h * att_ref[1, 2] + s_m * att_ref[2, 2]) * t_inv
    m = jnp.maximum(jnp.maximum(l0, l1), l2)
    e0 = jnp.exp(l0 - m)
    e1 = jnp.exp(l1 - m)
    e2 = jnp.exp(l2 - m)
    scale = 3.0 / (e0 + e1 + e2)
    out_ref[...] = (e0 * o_l + e1 * o_h + e2 * o_m) * scale


def kernel(input, adj_low, adj_high, adj_low_unnormalized,
           weight_low, weight_high, weight_mlp,
           att_vec_low, att_vec_high, att_vec_mlp, att_vec):
    del adj_low_unnormalized
    avl = att_vec_low.reshape(1, D)
    avh = att_vec_high.reshape(1, D)
    avm = att_vec_mlp.reshape(1, D)
    return pl.pallas_call(
        _fused_kernel,
        grid=(N // BM,),
        in_specs=[
            pl.BlockSpec((BM, N), lambda i: (i, 0)),
            pl.BlockSpec((BM, N), lambda i: (i, 0)),
            pl.BlockSpec((N, D), lambda i: (0, 0)),
            pl.BlockSpec((D, D), lambda i: (0, 0)),
            pl.BlockSpec((D, D), lambda i: (0, 0)),
            pl.BlockSpec((D, D), lambda i: (0, 0)),
            pl.BlockSpec((1, D), lambda i: (0, 0)),
            pl.BlockSpec((1, D), lambda i: (0, 0)),
            pl.BlockSpec((1, D), lambda i: (0, 0)),
            pl.BlockSpec(memory_space=pltpu.SMEM),
        ],
        out_specs=pl.BlockSpec((BM, D), lambda i: (i, 0)),
        out_shape=jax.ShapeDtypeStruct((N, D), jnp.float32),
        compiler_params=pltpu.CompilerParams(
            dimension_semantics=("arbitrary",)),
    )(adj_low, adj_high, input, weight_low, weight_high, weight_mlp,
      avl, avh, avm, att_vec)


# speedup vs baseline: 1.0058x; 1.0058x over previous
import jax
import jax.numpy as jnp
from jax.experimental import pallas as pl
from jax.experimental.pallas import tpu as pltpu

N = 4096
D = 128
BM = 512


def _fused_kernel(adj_low_ref, adj_high_ref, x_ref, wl_ref, wh_ref, wm_ref,
                  avl_ref, avh_ref, avm_ref, att_ref,
                  out_ref, xwl_ref, xwh_ref):
    i = pl.program_id(0)

    @pl.when(i == 0)
    def _precompute():
        x = x_ref[...]
        xwl_ref[...] = jnp.dot(x, wl_ref[...],
                               preferred_element_type=jnp.float32)
        xwh_ref[...] = jnp.dot(x, wh_ref[...],
                               preferred_element_type=jnp.float32)

    o_l = jax.nn.relu(jnp.dot(adj_low_ref[...], xwl_ref[...],
                              preferred_element_type=jnp.float32))
    o_h = jax.nn.relu(jnp.dot(adj_high_ref[...], xwh_ref[...],
                              preferred_element_type=jnp.float32))
    x_blk = x_ref[pl.ds(i * BM, BM), :]
    o_m = jax.nn.relu(jnp.dot(x_blk, wm_ref[...],
                              preferred_element_type=jnp.float32))

    f_l = jnp.sum(o_l * avl_ref[...], axis=1, keepdims=True)
    f_h = jnp.sum(o_h * avh_ref[...], axis=1, keepdims=True)
    f_m = jnp.sum(o_m * avm_ref[...], axis=1, keepdims=True)
    s_l = jax.nn.sigmoid(f_l)
    s_h = jax.nn.sigmoid(f_h)
    s_m = jax.nn.sigmoid(f_m)
    t_inv = 1.0 / 3.0
    l0 = (s_l * att_ref[0, 0] + s_h * att_ref[1, 0] + s_m * att_ref[2, 0]) * t_inv
    l1 = (s_l * att_ref[0, 1] + s_h * att_ref[1, 1] + s_m * att_ref[2, 1]) * t_inv
    l2 = (s_l * att_ref[0, 2] + s_h * att_ref[1, 2] + s_m * att_ref[2, 2]) * t_inv
    m = jnp.maximum(jnp.maximum(l0, l1), l2)
    e0 = jnp.exp(l0 - m)
    e1 = jnp.exp(l1 - m)
    e2 = jnp.exp(l2 - m)
    scale = 3.0 / (e0 + e1 + e2)
    out_ref[...] = (e0 * o_l + e1 * o_h + e2 * o_m) * scale


def kernel(input, adj_low, adj_high, adj_low_unnormalized,
           weight_low, weight_high, weight_mlp,
           att_vec_low, att_vec_high, att_vec_mlp, att_vec):
    del adj_low_unnormalized
    avl = att_vec_low.reshape(1, D)
    avh = att_vec_high.reshape(1, D)
    avm = att_vec_mlp.reshape(1, D)
    return pl.pallas_call(
        _fused_kernel,
        grid=(N // BM,),
        in_specs=[
            pl.BlockSpec((BM, N), lambda i: (i, 0)),
            pl.BlockSpec((BM, N), lambda i: (i, 0)),
            pl.BlockSpec((N, D), lambda i: (0, 0)),
            pl.BlockSpec((D, D), lambda i: (0, 0)),
            pl.BlockSpec((D, D), lambda i: (0, 0)),
            pl.BlockSpec((D, D), lambda i: (0, 0)),
            pl.BlockSpec((1, D), lambda i: (0, 0)),
            pl.BlockSpec((1, D), lambda i: (0, 0)),
            pl.BlockSpec((1, D), lambda i: (0, 0)),
            pl.BlockSpec(memory_space=pltpu.SMEM),
        ],
        out_specs=pl.BlockSpec((BM, D), lambda i: (i, 0)),
        out_shape=jax.ShapeDtypeStruct((N, D), jnp.float32),
        scratch_shapes=[
            pltpu.VMEM((N, D), jnp.float32),
            pltpu.VMEM((N, D), jnp.float32),
        ],
        compiler_params=pltpu.CompilerParams(
            dimension_semantics=("arbitrary",)),
    )(adj_low, adj_high, input, weight_low, weight_high, weight_mlp,
      avl, avh, avm, att_vec)
